# Initial kernel scaffold; baseline (speedup 1.0000x reference)
import jax
import jax.numpy as jnp
from jax.experimental import pallas as pl


def kernel(x, batch):
    raise NotImplementedError("write your pallas kernel here")


# speedup vs baseline: 3.3737x; 3.3737x over previous
import functools

import jax
import jax.numpy as jnp
from jax import lax
from jax.experimental import pallas as pl
from jax.experimental.pallas import tpu as pltpu
from jax.experimental.pallas import tpu_sc as plsc

N = 160000
D = 256
S = 10000
L = 16
CH = D // L
NC = 2
NS = 16
NW = NC * NS
RPW = N // NW
C = 64
BS_ITERS = 18
NEG_INF = float("-inf")


def _pool_body(x_hbm, b_hbm, out_hbm, xbuf, idbuf, headbuf, tailbuf, probebuf,
               stage, zeros):
    wid = lax.axis_index("s") * NC + lax.axis_index("c")
    r0 = wid * RPW
    r1 = r0 + RPW

    @pl.when(wid > 0)
    def _():
        pltpu.sync_copy(b_hbm.at[pl.ds(pl.multiple_of(r0 - L, 8), L)],
                        headbuf)

    @pl.when(wid == 0)
    def _():
        headbuf[...] = jnp.full((L,), -1, jnp.int32)

    pltpu.sync_copy(b_hbm.at[pl.ds(pl.multiple_of(r1 - L, 8), L)], tailbuf)

    prev_seg = headbuf[...][L - 1]
    last_seg = tailbuf[...][L - 1]
    zlo = prev_seg + 1
    zhi = jnp.where(wid == NW - 1, S - 1, last_seg)

    for j in range(2 * CH):
        zeros[pl.ds(j * L, L)] = jnp.zeros((L,), jnp.float32)

    not_last = wid < NW - 1

    @pl.when(not_last)
    def _():
        pltpu.sync_copy(b_hbm.at[pl.ds(pl.multiple_of(r1, 8), L)],
                        probebuf.at[pl.ds(0, L)])

    peek = probebuf[pl.ds(0, L)]
    lane = lax.iota(jnp.int32, L)
    ffs = jnp.min(jnp.where(peek != last_seg, lane, jnp.int32(L)))
    found = ffs < L
    bs_lo = jnp.where(found, r1 + ffs - 1, r1 + L - 1)
    bs_hi = jnp.where(found, r1 + ffs, N)
    bs_lo = jnp.where(not_last, bs_lo, N - 1)
    bs_hi = jnp.where(not_last, bs_hi, N)

    def bs_body(t, carry):
        lo, hi = carry
        act = (hi - lo) > 1
        mid = lax.div(lo + hi, 2)
        base = jnp.minimum(mid & ~jnp.int32(7), N - L)
        off = mid - base

        @pl.when(act)
        def _():
            pltpu.sync_copy(b_hbm.at[pl.ds(pl.multiple_of(base, 8), L)],
                            probebuf.at[pl.ds(0, L)])

        v = probebuf[pl.ds(off, L)][0]
        go_right = v == last_seg
        lo = jnp.where(jnp.logical_and(act, go_right), mid, lo)
        hi = jnp.where(jnp.logical_and(act, jnp.logical_not(go_right)),
                       mid, hi)
        return lo, hi

    _, re = lax.fori_loop(0, BS_ITERS, bs_body, (bs_lo, bs_hi))

    def zero_fill(lo, hi):
        lo = jnp.maximum(lo, zlo)

        def zbody(g, carry):
            pltpu.sync_copy(zeros, out_hbm.at[g])
            return carry

        lax.fori_loop(lo, jnp.maximum(lo, hi), zbody, 0)

    def flush(s_old, cnt, sums, maxs):
        @pl.when(s_old >= zlo)
        def _():
            inv = 1.0 / jnp.full((L,), cnt, jnp.float32)
            for j in range(CH):
                stage[pl.ds(j * L, L)] = sums[j] * inv
                stage[pl.ds(D + j * L, L)] = maxs[j]
            pltpu.sync_copy(stage, out_hbm.at[s_old])

    zero_v = jnp.zeros((L,), jnp.float32)
    ninf_v = jnp.full((L,), NEG_INF, jnp.float32)
    init_sums = (zero_v,) * CH
    init_maxs = (ninf_v,) * CH

    k0 = r0 // C
    kend = lax.div(re + C - 1, jnp.int32(C))

    def chunk_step(k, state):
        cur_seg, cnt, sums, maxs = state
        c = k * C
        pltpu.sync_copy(x_hbm.at[pl.ds(c, C)], xbuf)
        pltpu.sync_copy(b_hbm.at[pl.ds(pl.multiple_of(c, 8), C)],
                        idbuf.at[pl.ds(0, C)])

        def row_body(i, carry):
            cur_seg, cnt, sums, maxs = carry
            g = c + i
            seg = idbuf[pl.ds(i, L)][0]

            def do_row(args):
                cur_seg, cnt, sums, maxs = args

                def do_flush(args):
                    cur_seg, cnt, sums, maxs = args
                    flush(cur_seg, cnt, sums, maxs)
                    zero_fill(cur_seg + 1, seg)
                    return seg, jnp.int32(0), init_sums, init_maxs

                cur_seg, cnt, sums, maxs = lax.cond(
                    seg != cur_seg, do_flush, lambda a: a,
                    (cur_seg, cnt, sums, maxs))
                new_sums = tuple(
                    sums[j] + xbuf[i, pl.ds(j * L, L)] for j in range(CH))
                new_maxs = tuple(
                    jnp.maximum(maxs[j], xbuf[i, pl.ds(j * L, L)])
                    for j in range(CH))
                return cur_seg, cnt + 1, new_sums, new_maxs

            return lax.cond(g < re, do_row, lambda a: a,
                            (cur_seg, cnt, sums, maxs))

        return lax.fori_loop(0, C, row_body, (cur_seg, cnt, sums, maxs))

    init = (prev_seg, jnp.int32(0), init_sums, init_maxs)
    cur_seg, cnt, sums, maxs = lax.fori_loop(k0, kend, chunk_step, init)

    flush(cur_seg, cnt, sums, maxs)
    zero_fill(cur_seg + 1, zhi + 1)


@jax.jit
def _pooling(x, batch):
    mesh = plsc.VectorSubcoreMesh(core_axis_name="c", subcore_axis_name="s")
    return pl.kernel(
        _pool_body,
        out_type=jax.ShapeDtypeStruct((S, 2 * D), jnp.float32),
        mesh=mesh,
        compiler_params=pltpu.CompilerParams(needs_layout_passes=False),
        scratch_types=[
            pltpu.VMEM((C, D), jnp.float32),
            pltpu.VMEM((C + L, ), jnp.int32),
            pltpu.VMEM((L,), jnp.int32),
            pltpu.VMEM((L,), jnp.int32),
            pltpu.VMEM((2 * L,), jnp.int32),
            pltpu.VMEM((2 * D,), jnp.float32),
            pltpu.VMEM((2 * D,), jnp.float32),
        ],
    )(x, batch)


def kernel(x, batch):
    return _pooling(x, batch.astype(jnp.int32))
